# baseline (speedup 1.0000x reference)
import functools

import jax
import jax.numpy as jnp
from jax import lax
from jax.experimental import pallas as pl
from jax.experimental.pallas import tpu as pltpu
from jax.experimental.pallas import tpu_sc as plsc

_SIZE = 512
_L = 16
_NC = 2
_NS = 16


@functools.lru_cache(maxsize=None)
def _make_project(T, P, NV, NU):
    tpad = -(-T // _L) * _L
    nchunk = P // _L

    mesh = plsc.VectorSubcoreMesh(
        core_axis_name="c", subcore_axis_name="s", num_cores=_NC, num_subcores=_NS
    )

    @functools.partial(
        pl.kernel,
        out_type=jax.ShapeDtypeStruct((P * 3,), jnp.float32),
        mesh=mesh,
        compiler_params=pltpu.CompilerParams(needs_layout_passes=False),
        scratch_types=[
            pltpu.VMEM((3 * NV,), jnp.float32),
            pltpu.VMEM((2 * NU,), jnp.float32),
            pltpu.VMEM((3 * T,), jnp.int32),
            pltpu.VMEM((3 * T,), jnp.int32),
            pltpu.VMEM((2 * _L,), jnp.float32),
            pltpu.VMEM((tpad * 32,), jnp.float32),
            pltpu.VMEM((3 * _L,), jnp.float32),
            pltpu.SemaphoreType.DMA,
            pltpu.SemaphoreType.DMA,
            pltpu.SemaphoreType.DMA,
            pltpu.SemaphoreType.DMA,
            pltpu.SemaphoreType.DMA,
        ],
    )
    def project(vert_hbm, uv_hbm, fac_hbm, ufac_hbm, pts_hbm, out_hbm,
                vert, uvv, fac, ufac, ptv, tab, obuf, s0, s1, s2, s3, s4):
        wid = lax.axis_index("s") * _NC + lax.axis_index("c")

        @pl.when(wid < nchunk)
        def _():
            d0 = pltpu.async_copy(vert_hbm, vert, s0)
            d1 = pltpu.async_copy(uv_hbm, uvv, s1)
            d2 = pltpu.async_copy(fac_hbm, fac, s2)
            d3 = pltpu.async_copy(ufac_hbm, ufac, s3)
            d4 = pltpu.async_copy(pts_hbm.at[pl.ds(wid * 2 * _L, 2 * _L)], ptv, s4)
            d0.wait()
            d1.wait()
            d2.wait()
            d3.wait()
            d4.wait()

            iota = lax.broadcasted_iota(jnp.int32, (_L,), 0)

            for g in range(tpad // _L):
                lt = iota + (g * _L)
                if (g + 1) * _L > T:
                    lt = jnp.minimum(lt, T - 1)
                fi0 = plsc.load_gather(fac, [lt * 3])
                fi1 = plsc.load_gather(fac, [lt * 3 + 1])
                fi2 = plsc.load_gather(fac, [lt * 3 + 2])
                uf0 = plsc.load_gather(ufac, [lt * 3])
                uf1 = plsc.load_gather(ufac, [lt * 3 + 1])
                uf2 = plsc.load_gather(ufac, [lt * 3 + 2])

                ax = plsc.load_gather(vert, [fi0 * 3])
                ay = plsc.load_gather(vert, [fi0 * 3 + 1])
                az = plsc.load_gather(vert, [fi0 * 3 + 2])
                bx = plsc.load_gather(vert, [fi1 * 3])
                by = plsc.load_gather(vert, [fi1 * 3 + 1])
                bz = plsc.load_gather(vert, [fi1 * 3 + 2])
                cx = plsc.load_gather(vert, [fi2 * 3])
                cy = plsc.load_gather(vert, [fi2 * 3 + 1])
                cz = plsc.load_gather(vert, [fi2 * 3 + 2])
                ua = plsc.load_gather(uvv, [uf0 * 2])
                va = plsc.load_gather(uvv, [uf0 * 2 + 1])
                ub = plsc.load_gather(uvv, [uf1 * 2])
                vb = plsc.load_gather(uvv, [uf1 * 2 + 1])
                uc = plsc.load_gather(uvv, [uf2 * 2])
                vc = plsc.load_gather(uvv, [uf2 * 2 + 1])

                cross = (bx - ax) * (cy - ay) - (by - ay) * (cx - ax)
                w = 0.5 * cross
                valid = (cross > 0.0) & (w >= 1e-9)
                wsafe = jnp.where(w == 0.0, 1.0, w)
                h = 0.5 / wsafe

                def edge(qx, qy, rx, ry):
                    return ((qx * ry - qy * rx) * h,
                            (qy - ry) * h,
                            (rx - qx) * h)

                w1c0, w1cx, w1cy = edge(bx, by, cx, cy)
                w2c0, w2cx, w2cy = edge(cx, cy, ax, ay)
                a0c0, a0cx, a0cy = edge(ax, ay, bx, by)

                inf = jnp.float32(jnp.inf)
                xmin = jnp.where(valid, jnp.minimum(jnp.minimum(ax, bx), cx), inf)
                xmax = jnp.where(valid, jnp.maximum(jnp.maximum(ax, bx), cx), -inf)
                ymin = jnp.minimum(jnp.minimum(ay, by), cy)
                ymax = jnp.maximum(jnp.maximum(ay, by), cy)

                zia = 1.0 / az
                zib = 1.0 / bz
                zic = 1.0 / cz
                rows = [
                    xmin, xmax, ymin, ymax,
                    w1c0, w1cx, w1cy,
                    w2c0, w2cx, w2cy,
                    a0c0, a0cx, a0cy,
                    ua * zia, ub * zib, uc * zic,
                    va * zia, vb * zib, vc * zic,
                    zia, zib, zic,
                ]
                lanes = iota + (g * _L)
                for k, val in enumerate(rows):
                    plsc.store_scatter(tab, [lanes * 32 + k], val)

            px = plsc.load_gather(ptv, [iota * 2])
            py = plsc.load_gather(ptv, [iota * 2 + 1])
            px = px / (_SIZE - 1) * 2.0 - 1.0
            py = (_SIZE - py) / (_SIZE - 1) * 2.0 - 1.0

            bs = jnp.full((_L,), -jnp.inf, jnp.float32)
            bu = jnp.zeros((_L,), jnp.float32)
            bv = jnp.zeros((_L,), jnp.float32)
            for t in range(T):
                ca = tab[pl.ds(t * 32, _L)]
                cb = tab[pl.ds(t * 32 + _L, _L)]
                inb = ((px >= ca[0]) & (px <= ca[1])
                       & (py >= ca[2]) & (py <= ca[3]))
                w1 = ca[4] + ca[5] * px + ca[6] * py
                w2 = ca[7] + ca[8] * px + ca[9] * py
                a0 = ca[10] + ca[11] * px + ca[12] * py
                w3 = 1.0 - w1 - w2
                zi = w1 * cb[3] + w2 * cb[4] + w3 * cb[5]
                ptz = 1.0 / zi
                uu = (w1 * ca[13] + w2 * ca[14] + w3 * ca[15]) * ptz
                vv = (w1 * cb[0] + w2 * cb[1] + w3 * cb[2]) * ptz
                upd = (inb & (w1 >= 0.0) & (w2 >= 0.0) & (a0 >= 0.0)
                       & (ptz > bs))
                bs = jnp.where(upd, ptz, bs)
                bu = jnp.where(upd, uu, bu)
                bv = jnp.where(upd, vv, bv)

            plsc.store_scatter(obuf, [iota * 3], bu)
            plsc.store_scatter(obuf, [iota * 3 + 1], bv)
            plsc.store_scatter(obuf, [iota * 3 + 2], bs)
            pltpu.sync_copy(obuf, out_hbm.at[pl.ds(wid * 3 * _L, 3 * _L)])

    return project


def kernel(vertices, points, faces, uv, uvfaces):
    T = faces.shape[0]
    P = points.shape[0]
    NV = vertices.shape[0]
    NU = uv.shape[0]

    out = _make_project(T, P, NV, NU)(
        vertices.reshape(-1),
        uv.reshape(-1),
        faces.astype(jnp.int32).reshape(-1),
        uvfaces.astype(jnp.int32).reshape(-1),
        points.reshape(-1),
    )
    return out.reshape(P, 3)
